# baseline (speedup 1.0000x reference)
import functools

import jax
import jax.numpy as jnp
from jax import lax
from jax.experimental import pallas as pl
from jax.experimental.pallas import tpu as pltpu
from jax.experimental.pallas import tpu_sc as plsc

B = 64
S = 448
E = 1024
NC = 2
NS = 16
NW = NC * NS
SPW = S // NW
LANES = 16
GROUPS = (SPW * E) // LANES


def _body(ids_hbm, wte_hbm, wpe_hbm, out_hbm, idx_v, wpe_v, buf_v, gsem):
    w = lax.axis_index("s") * NC + lax.axis_index("c")
    pltpu.sync_copy(ids_hbm.at[w], idx_v)
    pltpu.sync_copy(wpe_hbm.at[w], wpe_v)

    def batch_step(b, carry):
        pltpu.async_copy(wte_hbm.at[idx_v.at[b]], buf_v, gsem).wait()
        for p in range(SPW):
            def add_grp(j, c):
                sl = pl.ds(j * LANES, LANES)
                buf_v[p, sl] = buf_v[p, sl] + wpe_v[p, sl]
                return c
            lax.fori_loop(0, E // LANES, add_grp, 0)
        pltpu.sync_copy(buf_v, out_hbm.at[b, w])
        return carry

    lax.fori_loop(0, B, batch_step, 0)


@jax.jit
def kernel(input_ids, wte, wpe):
    ids = input_ids.astype(jnp.int32)
    ids_prep = ids.reshape(B, NW, SPW).transpose(1, 0, 2)
    wpe_prep = wpe.reshape(NW, SPW, E)
    run = pl.kernel(
        _body,
        out_type=jax.ShapeDtypeStruct((B, NW, SPW, E), jnp.float32),
        mesh=plsc.VectorSubcoreMesh(core_axis_name="c", subcore_axis_name="s"),
        compiler_params=pltpu.CompilerParams(use_tc_tiling_on_sc=False),
        scratch_types=[
            pltpu.VMEM((B, SPW), jnp.int32),
            pltpu.VMEM((SPW, E), jnp.float32),
            pltpu.VMEM((SPW, E), jnp.float32),
            pltpu.SemaphoreType.DMA,
        ],
    )
    out = run(ids_prep, wte, wpe_prep)
    return out.reshape(B, S, E)


# speedup vs baseline: 2.9912x; 2.9912x over previous
import functools

import jax
import jax.numpy as jnp
from jax import lax
from jax.experimental import pallas as pl
from jax.experimental.pallas import tpu as pltpu
from jax.experimental.pallas import tpu_sc as plsc

B = 64
S = 448
E = 1024
NC = 2
NS = 16
SPW = 16
NACT = S // SPW
LANES = 16
NBUF = 2


def _body(ids_hbm, wte_hbm, wpe_hbm, out_hbm, idx_v, wpe_v, in_v, out_v,
          gsems, osems):
    w = lax.axis_index("s") * NC + lax.axis_index("c")

    @pl.when(w < NACT)
    def _work():
        s0 = w * SPW
        pltpu.sync_copy(ids_hbm.at[pl.ds(w * (B * SPW), B * SPW)], idx_v)
        pltpu.sync_copy(wpe_hbm.at[pl.ds(s0, SPW), :], wpe_v)

        for i in range(NBUF):
            pltpu.async_copy(
                wte_hbm.at[idx_v.at[pl.ds(i * SPW, SPW)]], in_v.at[i],
                gsems.at[i])

        def outer(g, carry):
            b0 = g * NBUF
            for i in range(NBUF):
                b = b0 + i
                pltpu.make_async_copy(
                    wte_hbm.at[idx_v.at[pl.ds(b * SPW, SPW)]], in_v.at[i],
                    gsems.at[i]).wait()
                @pl.when(g > 0)
                def _():
                    pltpu.make_async_copy(
                        out_v.at[i], out_hbm.at[b - NBUF, pl.ds(s0, SPW), :],
                        osems.at[i]).wait()
                for p in range(SPW):
                    def add_grp(j, c):
                        sl = pl.ds(j * LANES, LANES)
                        out_v[i, p, sl] = in_v[i, p, sl] + wpe_v[p, sl]
                        return c
                    lax.fori_loop(0, E // LANES, add_grp, 0, unroll=4)
                @pl.when(b + NBUF < B)
                def _():
                    pltpu.async_copy(
                        wte_hbm.at[idx_v.at[pl.ds((b + NBUF) * SPW, SPW)]],
                        in_v.at[i], gsems.at[i])
                pltpu.async_copy(
                    out_v.at[i], out_hbm.at[b, pl.ds(s0, SPW), :], osems.at[i])
            return carry

        lax.fori_loop(0, B // NBUF, outer, 0)

        for i in range(NBUF):
            b = B - NBUF + i
            pltpu.make_async_copy(
                out_v.at[i], out_hbm.at[b, pl.ds(s0, SPW), :],
                osems.at[i]).wait()


@jax.jit
def kernel(input_ids, wte, wpe):
    ids = input_ids.astype(jnp.int32)
    ids_prep = ids.reshape(B, NACT, SPW).transpose(1, 0, 2).reshape(-1)
    run = pl.kernel(
        _body,
        out_type=jax.ShapeDtypeStruct((B, S, E), jnp.float32),
        mesh=plsc.VectorSubcoreMesh(core_axis_name="c", subcore_axis_name="s"),
        scratch_types=[
            pltpu.VMEM((B * SPW,), jnp.int32),
            pltpu.VMEM((SPW, E), jnp.float32),
            pltpu.VMEM((NBUF, SPW, E), jnp.float32),
            pltpu.VMEM((NBUF, SPW, E), jnp.float32),
            pltpu.SemaphoreType.DMA((NBUF,)),
            pltpu.SemaphoreType.DMA((NBUF,)),
        ],
    )
    return run(ids_prep, wte, wpe)


# speedup vs baseline: 5.9245x; 1.9806x over previous
import functools

import jax
import jax.numpy as jnp
from jax import lax
from jax.experimental import pallas as pl
from jax.experimental.pallas import tpu as pltpu
from jax.experimental.pallas import tpu_sc as plsc

B = 64
S = 448
E = 1024
NC = 2
NS = 16
SPW = 16
NACT = S // SPW
LANES = 16
GROUPS = (SPW * E) // LANES


def _body(ids_hbm, wte_hbm, wpe_hbm, out_hbm, idx_v, wpe_v, buf_v, gsems,
          osems):
    w = lax.axis_index("s") * NC + lax.axis_index("c")

    @pl.when(w < NACT)
    def _work():
        s0 = w * SPW

        def gather(b, slot):
            return pltpu.make_async_copy(
                wte_hbm.at[idx_v.at[pl.ds(b * SPW, SPW)]], buf_v.at[slot],
                gsems.at[slot])

        def store(b, slot):
            return pltpu.make_async_copy(
                buf_v.at[slot], out_hbm.at[b, pl.ds(s0, SPW), :],
                osems.at[slot])

        pltpu.sync_copy(ids_hbm.at[pl.ds(w * (B * SPW), B * SPW)], idx_v)
        pltpu.sync_copy(wpe_hbm.at[pl.ds(s0, SPW), :], wpe_v)

        gather(0, 0).start()
        gather(1, 1).start()

        def round_(t, carry):
            c0 = t * 4
            for ps in range(2):
                c = c0 + 2 * ps
                i0, i1 = 2 * ps, 2 * ps + 1
                j0, j1 = (i0 + 2) % 4, (i1 + 2) % 4
                gather(c, i0).wait()
                gather(c + 1, i1).wait()

                @plsc.parallel_loop(0, GROUPS, unroll=8)
                def _add(g):
                    p = lax.shift_right_logical(g, 6)
                    sl = pl.ds((g & 63) * LANES, LANES)
                    wv = wpe_v[p, sl]
                    buf_v[i0, p, sl] = buf_v[i0, p, sl] + wv
                    buf_v[i1, p, sl] = buf_v[i1, p, sl] + wv

                store(c, i0).start()
                store(c + 1, i1).start()

                @pl.when(c >= 2)
                def _():
                    store(c - 2, j0).wait()
                    store(c - 1, j1).wait()

                @pl.when(c + 2 < B)
                def _():
                    gather(c + 2, j0).start()
                    gather(c + 3, j1).start()

            return carry

        lax.fori_loop(0, B // 4, round_, 0)

        store(B - 2, 2).wait()
        store(B - 1, 3).wait()


@jax.jit
def kernel(input_ids, wte, wpe):
    ids = input_ids.astype(jnp.int32)
    ids_prep = ids.reshape(B, NACT, SPW).transpose(1, 0, 2).reshape(-1)
    run = pl.kernel(
        _body,
        out_type=jax.ShapeDtypeStruct((B, S, E), jnp.float32),
        mesh=plsc.VectorSubcoreMesh(core_axis_name="c", subcore_axis_name="s"),
        scratch_types=[
            pltpu.VMEM((B * SPW,), jnp.int32),
            pltpu.VMEM((SPW, E), jnp.float32),
            pltpu.VMEM((4, SPW, E), jnp.float32),
            pltpu.SemaphoreType.DMA((4,)),
            pltpu.SemaphoreType.DMA((4,)),
        ],
    )
    return run(ids_prep, wte, wpe)


# speedup vs baseline: 6.4693x; 1.0920x over previous
import functools

import jax
import jax.numpy as jnp
from jax import lax
from jax.experimental import pallas as pl
from jax.experimental.pallas import tpu as pltpu
from jax.experimental.pallas import tpu_sc as plsc

B = 64
S = 448
E = 1024
NC = 2
NS = 16
NW = NC * NS
SPW = 16
NCHUNK = S // SPW
NPAIR = B // 2
TPW = (NCHUNK * NPAIR) // NW
LANES = 16
GROUPS = (SPW * E) // LANES


def _body(ids_hbm, wte_hbm, wpe_hbm, out_hbm, idx_v, wpe_v, buf_v, gsems,
          osems):
    w = lax.axis_index("s") * NC + lax.axis_index("c")
    t0 = w * TPW

    pltpu.sync_copy(ids_hbm.at[pl.ds(t0 * 32, TPW * 32)], idx_v)

    def task_parts(tl):
        tau = t0 + tl
        c = lax.shift_right_logical(tau, 5)
        b = (tau & (NPAIR - 1)) * 2
        coff = pl.multiple_of(c * SPW, SPW)
        return c, coff, b

    def gather(tl, r, slot):
        return pltpu.make_async_copy(
            wte_hbm.at[idx_v.at[pl.ds((tl * 2 + r) * SPW, SPW)]],
            buf_v.at[slot], gsems.at[slot])

    def store(tl, r, slot):
        _, coff, b = task_parts(tl)
        return pltpu.make_async_copy(
            buf_v.at[slot], out_hbm.at[b + r, pl.ds(coff, SPW), :],
            osems.at[slot])

    def load_wpe(coff):
        pltpu.sync_copy(wpe_hbm.at[pl.ds(coff, SPW), :], wpe_v)

    c_first, coff_first, _ = task_parts(0)
    load_wpe(coff_first)
    gather(0, 0, 0).start()
    gather(0, 1, 1).start()

    def round_(rnd, c_prev):
        for ps in range(2):
            tl = rnd * 2 + ps
            i0, i1 = 2 * ps, 2 * ps + 1
            j0, j1 = (i0 + 2) % 4, (i1 + 2) % 4

            c, coff, b = task_parts(tl)

            @pl.when(c != c_prev)
            def _():
                load_wpe(coff)

            gather(tl, 0, i0).wait()
            gather(tl, 1, i1).wait()

            @plsc.parallel_loop(0, GROUPS, unroll=8)
            def _add(g):
                p = lax.shift_right_logical(g, 6)
                sl = pl.ds((g & 63) * LANES, LANES)
                wv = wpe_v[p, sl]
                buf_v[i0, p, sl] = buf_v[i0, p, sl] + wv
                buf_v[i1, p, sl] = buf_v[i1, p, sl] + wv

            store(tl, 0, i0).start()
            store(tl, 1, i1).start()

            @pl.when(tl >= 1)
            def _():
                store(tl - 1, 0, j0).wait()
                store(tl - 1, 1, j1).wait()

            @pl.when(tl + 1 < TPW)
            def _():
                gather(tl + 1, 0, j0).start()
                gather(tl + 1, 1, j1).start()

            c_prev = c
        return c_prev

    lax.fori_loop(0, TPW // 2, round_, c_first)

    store(TPW - 1, 0, 2).wait()
    store(TPW - 1, 1, 3).wait()


@jax.jit
def kernel(input_ids, wte, wpe):
    ids = input_ids.astype(jnp.int32)
    ids_prep = (ids.reshape(NPAIR, 2, NCHUNK, SPW)
                .transpose(2, 0, 1, 3).reshape(-1))
    run = pl.kernel(
        _body,
        out_type=jax.ShapeDtypeStruct((B, S, E), jnp.float32),
        mesh=plsc.VectorSubcoreMesh(core_axis_name="c", subcore_axis_name="s"),
        scratch_types=[
            pltpu.VMEM((TPW * 32,), jnp.int32),
            pltpu.VMEM((SPW, E), jnp.float32),
            pltpu.VMEM((4, SPW, E), jnp.float32),
            pltpu.SemaphoreType.DMA((4,)),
            pltpu.SemaphoreType.DMA((4,)),
        ],
    )
    return run(ids_prep, wte, wpe)


# speedup vs baseline: 6.4816x; 1.0019x over previous
import functools

import jax
import jax.numpy as jnp
from jax import lax
from jax.experimental import pallas as pl
from jax.experimental.pallas import tpu as pltpu
from jax.experimental.pallas import tpu_sc as plsc

B = 64
S = 448
E = 1024
NC = 2
NS = 16
NW = NC * NS
SPW = 16
NCHUNK = S // SPW
NPAIR = B // 2
TPW = (NCHUNK * NPAIR) // NW
LANES = 16
GROUPS = (SPW * E) // LANES


def _body(ids_hbm, wte_hbm, wpe_hbm, out_hbm, idx_v, wpe_v, buf_v, gsems,
          osems):
    w = lax.axis_index("s") * NC + lax.axis_index("c")
    t0 = w * TPW

    pltpu.sync_copy(ids_hbm.at[pl.ds(t0 * 32, TPW * 32)], idx_v)

    def task_parts(tl):
        tau = t0 + tl
        c = lax.shift_right_logical(tau, 5)
        b = (tau & (NPAIR - 1)) * 2
        coff = pl.multiple_of(c * SPW, SPW)
        return c, coff, b

    def gather(tl, r, slot):
        return pltpu.make_async_copy(
            wte_hbm.at[idx_v.at[pl.ds((tl * 2 + r) * SPW, SPW)]],
            buf_v.at[slot], gsems.at[slot])

    def store(tl, ps):
        _, coff, b = task_parts(tl)
        return pltpu.make_async_copy(
            buf_v.at[pl.ds(2 * ps, 2)],
            out_hbm.at[pl.ds(b, 2), pl.ds(coff, SPW), :],
            osems.at[ps])

    def load_wpe(coff):
        pltpu.sync_copy(wpe_hbm.at[pl.ds(coff, SPW), :], wpe_v)

    c_first, coff_first, _ = task_parts(0)
    load_wpe(coff_first)
    gather(0, 0, 0).start()
    gather(0, 1, 1).start()

    def round_(rnd, c_prev):
        for ps in range(2):
            tl = rnd * 2 + ps
            i0, i1 = 2 * ps, 2 * ps + 1
            j0, j1 = (i0 + 2) % 4, (i1 + 2) % 4

            c, coff, b = task_parts(tl)

            @pl.when(c != c_prev)
            def _():
                load_wpe(coff)

            gather(tl, 0, i0).wait()
            gather(tl, 1, i1).wait()

            @plsc.parallel_loop(0, GROUPS, unroll=16)
            def _add(g):
                p = lax.shift_right_logical(g, 6)
                sl = pl.ds((g & 63) * LANES, LANES)
                wv = wpe_v[p, sl]
                buf_v[i0, p, sl] = buf_v[i0, p, sl] + wv
                buf_v[i1, p, sl] = buf_v[i1, p, sl] + wv

            store(tl, ps).start()

            @pl.when(tl >= 1)
            def _():
                store(tl - 1, 1 - ps).wait()

            @pl.when(tl + 1 < TPW)
            def _():
                gather(tl + 1, 0, j0).start()
                gather(tl + 1, 1, j1).start()

            c_prev = c
        return c_prev

    lax.fori_loop(0, TPW // 2, round_, c_first)

    store(TPW - 1, 1).wait()


@jax.jit
def kernel(input_ids, wte, wpe):
    ids = input_ids.astype(jnp.int32)
    ids_prep = (ids.reshape(NPAIR, 2, NCHUNK, SPW)
                .transpose(2, 0, 1, 3).reshape(-1))
    run = pl.kernel(
        _body,
        out_type=jax.ShapeDtypeStruct((B, S, E), jnp.float32),
        mesh=plsc.VectorSubcoreMesh(core_axis_name="c", subcore_axis_name="s"),
        scratch_types=[
            pltpu.VMEM((TPW * 32,), jnp.int32),
            pltpu.VMEM((SPW, E), jnp.float32),
            pltpu.VMEM((4, SPW, E), jnp.float32),
            pltpu.SemaphoreType.DMA((4,)),
            pltpu.SemaphoreType.DMA((2,)),
        ],
    )
    return run(ids_prep, wte, wpe)
